# Initial kernel scaffold; baseline (speedup 1.0000x reference)
import jax
import jax.numpy as jnp
from jax.experimental import pallas as pl


def kernel(atoms_x, bonds_i, bonds_j):
    raise NotImplementedError("write your pallas kernel here")


# speedup vs baseline: 4.2189x; 4.2189x over previous
import jax
import jax.numpy as jnp
from jax import lax
from jax.experimental import pallas as pl
from jax.experimental.pallas import tpu as pltpu
from jax.experimental.pallas import tpu_sc as plsc

N_ATOMS = 50000
N_EDGES = 1600000
NUM_CORES = 2
NUM_SUBCORES = 16
NW = NUM_CORES * NUM_SUBCORES
EPW = N_EDGES // NW
CH = 80
BLK = 2000
NCH = BLK // CH
NBLK = EPW // BLK
GRP = CH // 16


def _sc_body(ax_hbm, ay_hbm, az_hbm, bi_hbm, bj_hbm, ovec_hbm, odist_hbm,
             idx_i_v, idx_j_v, gix_v, giy_v, giz_v, gjx_v, gjy_v, gjz_v,
             vec_v, dist_v, sem):
    wid = lax.axis_index("s") * NUM_CORES + lax.axis_index("c")
    iota16 = lax.iota(jnp.int32, 16)

    def block_body(b, carry):
        e0 = wid * EPW + b * BLK
        pltpu.sync_copy(bi_hbm.at[pl.ds(e0, BLK)], idx_i_v)
        pltpu.sync_copy(bj_hbm.at[pl.ds(e0, BLK)], idx_j_v)

        def chunk_body(j, carry2):
            ii = idx_i_v.at[pl.ds(j * CH, CH)]
            jj = idx_j_v.at[pl.ds(j * CH, CH)]
            cps = [
                pltpu.async_copy(ax_hbm.at[ii], gix_v, sem),
                pltpu.async_copy(ay_hbm.at[ii], giy_v, sem),
                pltpu.async_copy(az_hbm.at[ii], giz_v, sem),
                pltpu.async_copy(ax_hbm.at[jj], gjx_v, sem),
                pltpu.async_copy(ay_hbm.at[jj], gjy_v, sem),
                pltpu.async_copy(az_hbm.at[jj], gjz_v, sem),
            ]
            for cp in cps:
                cp.wait()
            for g in range(GRP):
                o = g * 16
                v0 = gjx_v[pl.ds(o, 16)] - gix_v[pl.ds(o, 16)]
                v1 = gjy_v[pl.ds(o, 16)] - giy_v[pl.ds(o, 16)]
                v2 = gjz_v[pl.ds(o, 16)] - giz_v[pl.ds(o, 16)]
                er3 = (j * CH + o) * 3 + iota16 * 3
                plsc.store_scatter(vec_v, [er3], v0)
                plsc.store_scatter(vec_v, [er3 + 1], v1)
                plsc.store_scatter(vec_v, [er3 + 2], v2)
                d2 = v0 * v0 + v1 * v1 + v2 * v2
                d2c = jnp.maximum(d2, 1.1754944e-38)
                y = plsc.bitcast(0x5F3759DF - (plsc.bitcast(d2c, jnp.int32) >> 1),
                                 jnp.float32)
                y = y * (1.5 - 0.5 * d2c * y * y)
                y = y * (1.5 - 0.5 * d2c * y * y)
                y = y * (1.5 - 0.5 * d2c * y * y)
                dist_v[pl.ds(j * CH + o, 16)] = d2 * y
            return carry2

        lax.fori_loop(0, NCH, chunk_body, 0, unroll=False)
        pltpu.sync_copy(vec_v, ovec_hbm.at[pl.ds(e0 * 3, BLK * 3)])
        pltpu.sync_copy(dist_v, odist_hbm.at[pl.ds(e0, BLK)])
        return carry

    lax.fori_loop(0, NBLK, block_body, 0, unroll=False)


@jax.jit
def _sc_call(ax, ay, az, bi, bj):
    mesh = plsc.VectorSubcoreMesh(core_axis_name="c", subcore_axis_name="s",
                                  num_cores=NUM_CORES,
                                  num_subcores=NUM_SUBCORES)
    f = pl.kernel(
        _sc_body,
        out_type=(jax.ShapeDtypeStruct((N_EDGES * 3,), jnp.float32),
                  jax.ShapeDtypeStruct((N_EDGES,), jnp.float32)),
        mesh=mesh,
        compiler_params=pltpu.CompilerParams(needs_layout_passes=False,
                                             use_tc_tiling_on_sc=False),
        scratch_types=[
            pltpu.VMEM((BLK,), jnp.int32),
            pltpu.VMEM((BLK,), jnp.int32),
            pltpu.VMEM((CH,), jnp.float32),
            pltpu.VMEM((CH,), jnp.float32),
            pltpu.VMEM((CH,), jnp.float32),
            pltpu.VMEM((CH,), jnp.float32),
            pltpu.VMEM((CH,), jnp.float32),
            pltpu.VMEM((CH,), jnp.float32),
            pltpu.VMEM((BLK * 3,), jnp.float32),
            pltpu.VMEM((BLK,), jnp.float32),
            pltpu.SemaphoreType.DMA,
        ],
    )
    return f(ax, ay, az, bi, bj)


def kernel(atoms_x, bonds_i, bonds_j):
    ax = atoms_x[:, 0]
    ay = atoms_x[:, 1]
    az = atoms_x[:, 2]
    vec_flat, dist = _sc_call(ax, ay, az,
                              bonds_i.astype(jnp.int32),
                              bonds_j.astype(jnp.int32))
    return vec_flat.reshape(N_EDGES, 3), dist


# speedup vs baseline: 4.8680x; 1.1538x over previous
import jax
import jax.numpy as jnp
from jax import lax
from jax.experimental import pallas as pl
from jax.experimental.pallas import tpu as pltpu
from jax.experimental.pallas import tpu_sc as plsc

N_ATOMS = 50000
N_EDGES = 1600000
NUM_CORES = 2
NUM_SUBCORES = 16
NW = NUM_CORES * NUM_SUBCORES
EPW = N_EDGES // NW
CH = 80
BLK = 2000
NCH = BLK // CH
NBLK = EPW // BLK
GRP = CH // 16


def _sc_body(ax_hbm, ay_hbm, az_hbm, bi_hbm, bj_hbm, ovec_hbm, odist_hbm,
             idx_i_v, idx_j_v, gix_v, giy_v, giz_v, gjx_v, gjy_v, gjz_v,
             vec_v, dist_v, sem):
    wid = lax.axis_index("s") * NUM_CORES + lax.axis_index("c")
    iota16 = lax.iota(jnp.int32, 16)
    col0 = iota16 * 0
    col1 = col0 + 1
    col2 = col0 + 2

    def block_body(b, carry):
        e0 = wid * EPW + b * BLK
        pltpu.sync_copy(bi_hbm.at[pl.ds(e0, BLK)], idx_i_v)
        pltpu.sync_copy(bj_hbm.at[pl.ds(e0, BLK)], idx_j_v)

        def chunk_body(j, carry2):
            ii = idx_i_v.at[pl.ds(j * CH, CH)]
            jj = idx_j_v.at[pl.ds(j * CH, CH)]
            cps = [
                pltpu.async_copy(ax_hbm.at[ii], gix_v, sem),
                pltpu.async_copy(ay_hbm.at[ii], giy_v, sem),
                pltpu.async_copy(az_hbm.at[ii], giz_v, sem),
                pltpu.async_copy(ax_hbm.at[jj], gjx_v, sem),
                pltpu.async_copy(ay_hbm.at[jj], gjy_v, sem),
                pltpu.async_copy(az_hbm.at[jj], gjz_v, sem),
            ]
            for cp in cps:
                cp.wait()
            for g in range(GRP):
                o = g * 16
                v0 = gjx_v[pl.ds(o, 16)] - gix_v[pl.ds(o, 16)]
                v1 = gjy_v[pl.ds(o, 16)] - giy_v[pl.ds(o, 16)]
                v2 = gjz_v[pl.ds(o, 16)] - giz_v[pl.ds(o, 16)]
                er = j * CH + o + iota16
                plsc.store_scatter(vec_v, [er, col0], v0)
                plsc.store_scatter(vec_v, [er, col1], v1)
                plsc.store_scatter(vec_v, [er, col2], v2)
                d2 = v0 * v0 + v1 * v1 + v2 * v2
                d2c = jnp.maximum(d2, 1.1754944e-38)
                y = plsc.bitcast(0x5F3759DF - (plsc.bitcast(d2c, jnp.int32) >> 1),
                                 jnp.float32)
                y = y * (1.5 - 0.5 * d2c * y * y)
                y = y * (1.5 - 0.5 * d2c * y * y)
                y = y * (1.5 - 0.5 * d2c * y * y)
                dist_v[pl.ds(j * CH + o, 16)] = d2 * y
            return carry2

        lax.fori_loop(0, NCH, chunk_body, 0, unroll=False)
        pltpu.sync_copy(vec_v, ovec_hbm.at[pl.ds(e0, BLK)])
        pltpu.sync_copy(dist_v, odist_hbm.at[pl.ds(e0, BLK)])
        return carry

    lax.fori_loop(0, NBLK, block_body, 0, unroll=False)


@jax.jit
def _sc_call(ax, ay, az, bi, bj):
    mesh = plsc.VectorSubcoreMesh(core_axis_name="c", subcore_axis_name="s",
                                  num_cores=NUM_CORES,
                                  num_subcores=NUM_SUBCORES)
    f = pl.kernel(
        _sc_body,
        out_type=(jax.ShapeDtypeStruct((N_EDGES, 3), jnp.float32),
                  jax.ShapeDtypeStruct((N_EDGES,), jnp.float32)),
        mesh=mesh,
        compiler_params=pltpu.CompilerParams(needs_layout_passes=False,
                                             use_tc_tiling_on_sc=False),
        scratch_types=[
            pltpu.VMEM((BLK,), jnp.int32),
            pltpu.VMEM((BLK,), jnp.int32),
            pltpu.VMEM((CH,), jnp.float32),
            pltpu.VMEM((CH,), jnp.float32),
            pltpu.VMEM((CH,), jnp.float32),
            pltpu.VMEM((CH,), jnp.float32),
            pltpu.VMEM((CH,), jnp.float32),
            pltpu.VMEM((CH,), jnp.float32),
            pltpu.VMEM((BLK, 3), jnp.float32),
            pltpu.VMEM((BLK,), jnp.float32),
            pltpu.SemaphoreType.DMA,
        ],
    )
    return f(ax, ay, az, bi, bj)


def kernel(atoms_x, bonds_i, bonds_j):
    ax = atoms_x[:, 0]
    ay = atoms_x[:, 1]
    az = atoms_x[:, 2]
    return _sc_call(ax, ay, az,
                    bonds_i.astype(jnp.int32),
                    bonds_j.astype(jnp.int32))


# speedup vs baseline: 8.0263x; 1.6488x over previous
import jax
import jax.numpy as jnp
from jax import lax
from jax.experimental import pallas as pl
from jax.experimental.pallas import tpu as pltpu
from jax.experimental.pallas import tpu_sc as plsc

N_ATOMS = 50000
N_EDGES = 1600000
NUM_CORES = 2
NUM_SUBCORES = 16
NW = NUM_CORES * NUM_SUBCORES
EPW = N_EDGES // NW
CH = 80
BLK = 2000
NCH = BLK // CH
NBLK = EPW // BLK
GRP = CH // 16


def _sc_body(ax_hbm, ay_hbm, az_hbm, bi_hbm, bj_hbm,
             ovx_hbm, ovy_hbm, ovz_hbm, odist_hbm,
             idx_i_v, idx_j_v, gix_v, giy_v, giz_v, gjx_v, gjy_v, gjz_v,
             vx_v, vy_v, vz_v, dist_v, sem):
    wid = lax.axis_index("s") * NUM_CORES + lax.axis_index("c")

    def block_body(b, carry):
        e0 = wid * EPW + b * BLK
        pltpu.sync_copy(bi_hbm.at[pl.ds(e0, BLK)], idx_i_v)
        pltpu.sync_copy(bj_hbm.at[pl.ds(e0, BLK)], idx_j_v)

        def chunk_body(j, carry2):
            ii = idx_i_v.at[pl.ds(j * CH, CH)]
            jj = idx_j_v.at[pl.ds(j * CH, CH)]
            cps = [
                pltpu.async_copy(ax_hbm.at[ii], gix_v, sem),
                pltpu.async_copy(ay_hbm.at[ii], giy_v, sem),
                pltpu.async_copy(az_hbm.at[ii], giz_v, sem),
                pltpu.async_copy(ax_hbm.at[jj], gjx_v, sem),
                pltpu.async_copy(ay_hbm.at[jj], gjy_v, sem),
                pltpu.async_copy(az_hbm.at[jj], gjz_v, sem),
            ]
            for cp in cps:
                cp.wait()
            for g in range(GRP):
                o = g * 16
                v0 = gjx_v[pl.ds(o, 16)] - gix_v[pl.ds(o, 16)]
                v1 = gjy_v[pl.ds(o, 16)] - giy_v[pl.ds(o, 16)]
                v2 = gjz_v[pl.ds(o, 16)] - giz_v[pl.ds(o, 16)]
                eo = j * CH + o
                vx_v[pl.ds(eo, 16)] = v0
                vy_v[pl.ds(eo, 16)] = v1
                vz_v[pl.ds(eo, 16)] = v2
                d2 = v0 * v0 + v1 * v1 + v2 * v2
                d2c = jnp.maximum(d2, 1.1754944e-38)
                y = plsc.bitcast(0x5F3759DF - (plsc.bitcast(d2c, jnp.int32) >> 1),
                                 jnp.float32)
                y = y * (1.5 - 0.5 * d2c * y * y)
                y = y * (1.5 - 0.5 * d2c * y * y)
                y = y * (1.5 - 0.5 * d2c * y * y)
                dist_v[pl.ds(eo, 16)] = d2 * y
            return carry2

        lax.fori_loop(0, NCH, chunk_body, 0, unroll=False)
        pltpu.sync_copy(vx_v, ovx_hbm.at[pl.ds(e0, BLK)])
        pltpu.sync_copy(vy_v, ovy_hbm.at[pl.ds(e0, BLK)])
        pltpu.sync_copy(vz_v, ovz_hbm.at[pl.ds(e0, BLK)])
        pltpu.sync_copy(dist_v, odist_hbm.at[pl.ds(e0, BLK)])
        return carry

    lax.fori_loop(0, NBLK, block_body, 0, unroll=False)


@jax.jit
def _sc_call(ax, ay, az, bi, bj):
    mesh = plsc.VectorSubcoreMesh(core_axis_name="c", subcore_axis_name="s",
                                  num_cores=NUM_CORES,
                                  num_subcores=NUM_SUBCORES)
    f = pl.kernel(
        _sc_body,
        out_type=(jax.ShapeDtypeStruct((N_EDGES,), jnp.float32),
                  jax.ShapeDtypeStruct((N_EDGES,), jnp.float32),
                  jax.ShapeDtypeStruct((N_EDGES,), jnp.float32),
                  jax.ShapeDtypeStruct((N_EDGES,), jnp.float32)),
        mesh=mesh,
        compiler_params=pltpu.CompilerParams(needs_layout_passes=False,
                                             use_tc_tiling_on_sc=False),
        scratch_types=[
            pltpu.VMEM((BLK,), jnp.int32),
            pltpu.VMEM((BLK,), jnp.int32),
            pltpu.VMEM((CH,), jnp.float32),
            pltpu.VMEM((CH,), jnp.float32),
            pltpu.VMEM((CH,), jnp.float32),
            pltpu.VMEM((CH,), jnp.float32),
            pltpu.VMEM((CH,), jnp.float32),
            pltpu.VMEM((CH,), jnp.float32),
            pltpu.VMEM((BLK,), jnp.float32),
            pltpu.VMEM((BLK,), jnp.float32),
            pltpu.VMEM((BLK,), jnp.float32),
            pltpu.VMEM((BLK,), jnp.float32),
            pltpu.SemaphoreType.DMA,
        ],
    )
    return f(ax, ay, az, bi, bj)


def kernel(atoms_x, bonds_i, bonds_j):
    ax = atoms_x[:, 0]
    ay = atoms_x[:, 1]
    az = atoms_x[:, 2]
    vx, vy, vz, dist = _sc_call(ax, ay, az,
                                bonds_i.astype(jnp.int32),
                                bonds_j.astype(jnp.int32))
    return jnp.stack([vx, vy, vz], axis=-1), dist


# speedup vs baseline: 13.9874x; 1.7427x over previous
import jax
import jax.numpy as jnp
from jax import lax
from jax.experimental import pallas as pl
from jax.experimental.pallas import tpu as pltpu
from jax.experimental.pallas import tpu_sc as plsc

N_ATOMS = 50000
N_EDGES = 1600000
NUM_CORES = 2
NUM_SUBCORES = 16
NW = NUM_CORES * NUM_SUBCORES
EPW = N_EDGES // NW
CH = 80
BLK = 2000
NCH = BLK // CH
NBLK = EPW // BLK
GRP = CH // 16


def _sc_body(ax_hbm, ay_hbm, az_hbm, bi_hbm, bj_hbm,
             ovx_hbm, ovy_hbm, ovz_hbm, odist_hbm,
             idx_i_v, idx_j_v, gix_v, giy_v, giz_v, gjx_v, gjy_v, gjz_v,
             vx_v, vy_v, vz_v, dist_v, sem):
    wid = lax.axis_index("s") * NUM_CORES + lax.axis_index("c")

    def block_body(b, carry):
        e0 = wid * EPW + b * BLK
        pltpu.sync_copy(bi_hbm.at[pl.ds(e0, BLK)], idx_i_v)
        pltpu.sync_copy(bj_hbm.at[pl.ds(e0, BLK)], idx_j_v)

        cps = [
            pltpu.async_copy(ax_hbm.at[idx_i_v], gix_v, sem),
            pltpu.async_copy(ay_hbm.at[idx_i_v], giy_v, sem),
            pltpu.async_copy(az_hbm.at[idx_i_v], giz_v, sem),
            pltpu.async_copy(ax_hbm.at[idx_j_v], gjx_v, sem),
            pltpu.async_copy(ay_hbm.at[idx_j_v], gjy_v, sem),
            pltpu.async_copy(az_hbm.at[idx_j_v], gjz_v, sem),
        ]
        for cp in cps:
            cp.wait()

        def group_body(g, carry2):
            o = g * 16
            v0 = gjx_v[pl.ds(o, 16)] - gix_v[pl.ds(o, 16)]
            v1 = gjy_v[pl.ds(o, 16)] - giy_v[pl.ds(o, 16)]
            v2 = gjz_v[pl.ds(o, 16)] - giz_v[pl.ds(o, 16)]
            vx_v[pl.ds(o, 16)] = v0
            vy_v[pl.ds(o, 16)] = v1
            vz_v[pl.ds(o, 16)] = v2
            d2 = v0 * v0 + v1 * v1 + v2 * v2
            d2c = jnp.maximum(d2, 1.1754944e-38)
            y = plsc.bitcast(0x5F3759DF - (plsc.bitcast(d2c, jnp.int32) >> 1),
                             jnp.float32)
            y = y * (1.5 - 0.5 * d2c * y * y)
            y = y * (1.5 - 0.5 * d2c * y * y)
            y = y * (1.5 - 0.5 * d2c * y * y)
            dist_v[pl.ds(o, 16)] = d2 * y
            return carry2

        lax.fori_loop(0, BLK // 16, group_body, 0, unroll=False)
        pltpu.sync_copy(vx_v, ovx_hbm.at[pl.ds(e0, BLK)])
        pltpu.sync_copy(vy_v, ovy_hbm.at[pl.ds(e0, BLK)])
        pltpu.sync_copy(vz_v, ovz_hbm.at[pl.ds(e0, BLK)])
        pltpu.sync_copy(dist_v, odist_hbm.at[pl.ds(e0, BLK)])
        return carry

    lax.fori_loop(0, NBLK, block_body, 0, unroll=False)


@jax.jit
def _sc_call(ax, ay, az, bi, bj):
    mesh = plsc.VectorSubcoreMesh(core_axis_name="c", subcore_axis_name="s",
                                  num_cores=NUM_CORES,
                                  num_subcores=NUM_SUBCORES)
    f = pl.kernel(
        _sc_body,
        out_type=(jax.ShapeDtypeStruct((N_EDGES,), jnp.float32),
                  jax.ShapeDtypeStruct((N_EDGES,), jnp.float32),
                  jax.ShapeDtypeStruct((N_EDGES,), jnp.float32),
                  jax.ShapeDtypeStruct((N_EDGES,), jnp.float32)),
        mesh=mesh,
        compiler_params=pltpu.CompilerParams(needs_layout_passes=False,
                                             use_tc_tiling_on_sc=False),
        scratch_types=[
            pltpu.VMEM((BLK,), jnp.int32),
            pltpu.VMEM((BLK,), jnp.int32),
            pltpu.VMEM((BLK,), jnp.float32),
            pltpu.VMEM((BLK,), jnp.float32),
            pltpu.VMEM((BLK,), jnp.float32),
            pltpu.VMEM((BLK,), jnp.float32),
            pltpu.VMEM((BLK,), jnp.float32),
            pltpu.VMEM((BLK,), jnp.float32),
            pltpu.VMEM((BLK,), jnp.float32),
            pltpu.VMEM((BLK,), jnp.float32),
            pltpu.VMEM((BLK,), jnp.float32),
            pltpu.VMEM((BLK,), jnp.float32),
            pltpu.SemaphoreType.DMA,
        ],
    )
    return f(ax, ay, az, bi, bj)


def kernel(atoms_x, bonds_i, bonds_j):
    ax = atoms_x[:, 0]
    ay = atoms_x[:, 1]
    az = atoms_x[:, 2]
    vx, vy, vz, dist = _sc_call(ax, ay, az,
                                bonds_i.astype(jnp.int32),
                                bonds_j.astype(jnp.int32))
    return jnp.stack([vx, vy, vz], axis=-1), dist


# speedup vs baseline: 14.5365x; 1.0393x over previous
import jax
import jax.numpy as jnp
from jax import lax
from jax.experimental import pallas as pl
from jax.experimental.pallas import tpu as pltpu
from jax.experimental.pallas import tpu_sc as plsc

N_ATOMS = 50000
N_EDGES = 1600000
NUM_CORES = 2
NUM_SUBCORES = 16
NW = NUM_CORES * NUM_SUBCORES
EPW = N_EDGES // NW
BLK = 2000
NBLK = EPW // BLK


def _sc_body(ax_hbm, ay_hbm, az_hbm, bi_hbm, bj_hbm,
             ovx_hbm, ovy_hbm, ovz_hbm, odist_hbm,
             ii0, ij0, ii1, ij1,
             g0x, g0y, g0z, g0jx, g0jy, g0jz,
             g1x, g1y, g1z, g1jx, g1jy, g1jz,
             o0x, o0y, o0z, o0d, o1x, o1y, o1z, o1d,
             sg0, sg1, si0, si1, so0, so1):
    wid = lax.axis_index("s") * NUM_CORES + lax.axis_index("c")
    base = wid * EPW

    idx = [(ii0, ij0), (ii1, ij1)]
    gath = [(g0x, g0y, g0z, g0jx, g0jy, g0jz),
            (g1x, g1y, g1z, g1jx, g1jy, g1jz)]
    outs = [(o0x, o0y, o0z, o0d), (o1x, o1y, o1z, o1d)]
    sgs = [sg0, sg1]
    sis = [si0, si1]
    sos = [so0, so1]

    def issue_idx(b):
        p = b % 2
        e0 = base + b * BLK
        return (pltpu.async_copy(bi_hbm.at[pl.ds(e0, BLK)], idx[p][0], sis[p]),
                pltpu.async_copy(bj_hbm.at[pl.ds(e0, BLK)], idx[p][1], sis[p]))

    def issue_gathers(b):
        p = b % 2
        gi, gj = idx[p]
        gx, gy, gz, gjx, gjy, gjz = gath[p]
        s = sgs[p]
        return (pltpu.async_copy(ax_hbm.at[gi], gx, s),
                pltpu.async_copy(ay_hbm.at[gi], gy, s),
                pltpu.async_copy(az_hbm.at[gi], gz, s),
                pltpu.async_copy(ax_hbm.at[gj], gjx, s),
                pltpu.async_copy(ay_hbm.at[gj], gjy, s),
                pltpu.async_copy(az_hbm.at[gj], gjz, s))

    def issue_outs(b):
        p = b % 2
        vx_v, vy_v, vz_v, dist_v = outs[p]
        e0 = base + b * BLK
        s = sos[p]
        return (pltpu.async_copy(vx_v, ovx_hbm.at[pl.ds(e0, BLK)], s),
                pltpu.async_copy(vy_v, ovy_hbm.at[pl.ds(e0, BLK)], s),
                pltpu.async_copy(vz_v, ovz_hbm.at[pl.ds(e0, BLK)], s),
                pltpu.async_copy(dist_v, odist_hbm.at[pl.ds(e0, BLK)], s))

    def compute(b):
        p = b % 2
        gx, gy, gz, gjx, gjy, gjz = gath[p]
        vx_v, vy_v, vz_v, dist_v = outs[p]

        def group_body(g, carry):
            o = g * 16
            v0 = gjx[pl.ds(o, 16)] - gx[pl.ds(o, 16)]
            v1 = gjy[pl.ds(o, 16)] - gy[pl.ds(o, 16)]
            v2 = gjz[pl.ds(o, 16)] - gz[pl.ds(o, 16)]
            vx_v[pl.ds(o, 16)] = v0
            vy_v[pl.ds(o, 16)] = v1
            vz_v[pl.ds(o, 16)] = v2
            d2 = v0 * v0 + v1 * v1 + v2 * v2
            d2c = jnp.maximum(d2, 1.1754944e-38)
            y = plsc.bitcast(0x5F3759DF - (plsc.bitcast(d2c, jnp.int32) >> 1),
                             jnp.float32)
            y = y * (1.5 - 0.5 * d2c * y * y)
            y = y * (1.5 - 0.5 * d2c * y * y)
            y = y * (1.5 - 0.5 * d2c * y * y)
            dist_v[pl.ds(o, 16)] = d2 * y
            return carry

        lax.fori_loop(0, BLK // 16, group_body, 0, unroll=False)

    idx_descs = {0: issue_idx(0)}
    for d in idx_descs[0]:
        d.wait()
    g_descs = {0: issue_gathers(0)}
    idx_descs[1] = issue_idx(1)
    out_descs = {}
    for b in range(NBLK):
        if b + 1 < NBLK:
            for d in idx_descs[b + 1]:
                d.wait()
            g_descs[b + 1] = issue_gathers(b + 1)
        for d in g_descs[b]:
            d.wait()
        if b + 2 < NBLK:
            idx_descs[b + 2] = issue_idx(b + 2)
        if b - 2 in out_descs:
            for d in out_descs[b - 2]:
                d.wait()
        compute(b)
        out_descs[b] = issue_outs(b)
    for d in out_descs[NBLK - 2]:
        d.wait()
    for d in out_descs[NBLK - 1]:
        d.wait()


@jax.jit
def _sc_call(ax, ay, az, bi, bj):
    mesh = plsc.VectorSubcoreMesh(core_axis_name="c", subcore_axis_name="s",
                                  num_cores=NUM_CORES,
                                  num_subcores=NUM_SUBCORES)
    fvec = pltpu.VMEM((BLK,), jnp.float32)
    ivec = pltpu.VMEM((BLK,), jnp.int32)
    f = pl.kernel(
        _sc_body,
        out_type=(jax.ShapeDtypeStruct((N_EDGES,), jnp.float32),
                  jax.ShapeDtypeStruct((N_EDGES,), jnp.float32),
                  jax.ShapeDtypeStruct((N_EDGES,), jnp.float32),
                  jax.ShapeDtypeStruct((N_EDGES,), jnp.float32)),
        mesh=mesh,
        compiler_params=pltpu.CompilerParams(needs_layout_passes=False,
                                             use_tc_tiling_on_sc=False),
        scratch_types=(
            [ivec] * 4 + [fvec] * 12 + [fvec] * 8
            + [pltpu.SemaphoreType.DMA] * 6
        ),
    )
    return f(ax, ay, az, bi, bj)


def kernel(atoms_x, bonds_i, bonds_j):
    ax = atoms_x[:, 0]
    ay = atoms_x[:, 1]
    az = atoms_x[:, 2]
    vx, vy, vz, dist = _sc_call(ax, ay, az,
                                bonds_i.astype(jnp.int32),
                                bonds_j.astype(jnp.int32))
    return jnp.stack([vx, vy, vz], axis=-1), dist


# speedup vs baseline: 27.7059x; 1.9059x over previous
import jax
import jax.numpy as jnp
from jax import lax
from jax.experimental import pallas as pl
from jax.experimental.pallas import tpu as pltpu
from jax.experimental.pallas import tpu_sc as plsc

N_ATOMS = 50000
N_EDGES = 1600000
NUM_CORES = 2
NUM_SUBCORES = 16
NW = NUM_CORES * NUM_SUBCORES
EPW = N_EDGES // NW
BLK = 2000
NBLK = EPW // BLK


def _sc_body(ax4_hbm, bi_hbm, bj_hbm,
             ovx_hbm, ovy_hbm, ovz_hbm, odist_hbm,
             ii0, ij0, ii1, ij1,
             g0i, g0j, g1i, g1j,
             o0x, o0y, o0z, o0d, o1x, o1y, o1z, o1d,
             sg0, sg1, si0, si1, so0, so1):
    wid = lax.axis_index("s") * NUM_CORES + lax.axis_index("c")
    base = wid * EPW
    iota16 = lax.iota(jnp.int32, 16)
    col0 = iota16 * 0
    col1 = col0 + 1
    col2 = col0 + 2

    idx = [(ii0, ij0), (ii1, ij1)]
    gath = [(g0i, g0j), (g1i, g1j)]
    outs = [(o0x, o0y, o0z, o0d), (o1x, o1y, o1z, o1d)]
    sgs = [sg0, sg1]
    sis = [si0, si1]
    sos = [so0, so1]

    def issue_idx(b):
        p = b % 2
        e0 = base + b * BLK
        return (pltpu.async_copy(bi_hbm.at[pl.ds(e0, BLK)], idx[p][0], sis[p]),
                pltpu.async_copy(bj_hbm.at[pl.ds(e0, BLK)], idx[p][1], sis[p]))

    def issue_gathers(b):
        p = b % 2
        gi, gj = idx[p]
        ri, rj = gath[p]
        s = sgs[p]
        return (pltpu.async_copy(ax4_hbm.at[gi], ri, s),
                pltpu.async_copy(ax4_hbm.at[gj], rj, s))

    def issue_outs(b):
        p = b % 2
        vx_v, vy_v, vz_v, dist_v = outs[p]
        e0 = base + b * BLK
        s = sos[p]
        return (pltpu.async_copy(vx_v, ovx_hbm.at[pl.ds(e0, BLK)], s),
                pltpu.async_copy(vy_v, ovy_hbm.at[pl.ds(e0, BLK)], s),
                pltpu.async_copy(vz_v, ovz_hbm.at[pl.ds(e0, BLK)], s),
                pltpu.async_copy(dist_v, odist_hbm.at[pl.ds(e0, BLK)], s))

    def compute(b):
        p = b % 2
        ri, rj = gath[p]
        vx_v, vy_v, vz_v, dist_v = outs[p]

        def group_body(g, carry):
            o = g * 16
            rows = iota16 + o
            v0 = plsc.load_gather(rj, [rows, col0]) - plsc.load_gather(ri, [rows, col0])
            v1 = plsc.load_gather(rj, [rows, col1]) - plsc.load_gather(ri, [rows, col1])
            v2 = plsc.load_gather(rj, [rows, col2]) - plsc.load_gather(ri, [rows, col2])
            vx_v[pl.ds(o, 16)] = v0
            vy_v[pl.ds(o, 16)] = v1
            vz_v[pl.ds(o, 16)] = v2
            d2 = v0 * v0 + v1 * v1 + v2 * v2
            d2c = jnp.maximum(d2, 1.1754944e-38)
            y = plsc.bitcast(0x5F3759DF - (plsc.bitcast(d2c, jnp.int32) >> 1),
                             jnp.float32)
            y = y * (1.5 - 0.5 * d2c * y * y)
            y = y * (1.5 - 0.5 * d2c * y * y)
            y = y * (1.5 - 0.5 * d2c * y * y)
            dist_v[pl.ds(o, 16)] = d2 * y
            return carry

        lax.fori_loop(0, BLK // 16, group_body, 0, unroll=False)

    idx_descs = {0: issue_idx(0)}
    for d in idx_descs[0]:
        d.wait()
    g_descs = {0: issue_gathers(0)}
    idx_descs[1] = issue_idx(1)
    out_descs = {}
    for b in range(NBLK):
        if b + 1 < NBLK:
            for d in idx_descs[b + 1]:
                d.wait()
            g_descs[b + 1] = issue_gathers(b + 1)
        for d in g_descs[b]:
            d.wait()
        if b + 2 < NBLK:
            idx_descs[b + 2] = issue_idx(b + 2)
        if b - 2 in out_descs:
            for d in out_descs[b - 2]:
                d.wait()
        compute(b)
        out_descs[b] = issue_outs(b)
    for d in out_descs[NBLK - 2]:
        d.wait()
    for d in out_descs[NBLK - 1]:
        d.wait()


@jax.jit
def _sc_call(ax4, bi, bj):
    mesh = plsc.VectorSubcoreMesh(core_axis_name="c", subcore_axis_name="s",
                                  num_cores=NUM_CORES,
                                  num_subcores=NUM_SUBCORES)
    fvec = pltpu.VMEM((BLK,), jnp.float32)
    ivec = pltpu.VMEM((BLK,), jnp.int32)
    rvec = pltpu.VMEM((BLK, 8), jnp.float32)
    f = pl.kernel(
        _sc_body,
        out_type=(jax.ShapeDtypeStruct((N_EDGES,), jnp.float32),
                  jax.ShapeDtypeStruct((N_EDGES,), jnp.float32),
                  jax.ShapeDtypeStruct((N_EDGES,), jnp.float32),
                  jax.ShapeDtypeStruct((N_EDGES,), jnp.float32)),
        mesh=mesh,
        compiler_params=pltpu.CompilerParams(needs_layout_passes=False,
                                             use_tc_tiling_on_sc=False),
        scratch_types=(
            [ivec] * 4 + [rvec] * 4 + [fvec] * 8
            + [pltpu.SemaphoreType.DMA] * 6
        ),
    )
    return f(ax4, bi, bj)


def kernel(atoms_x, bonds_i, bonds_j):
    ax4 = jnp.concatenate(
        [atoms_x, jnp.zeros((N_ATOMS, 5), jnp.float32)], axis=1)
    vx, vy, vz, dist = _sc_call(ax4,
                                bonds_i.astype(jnp.int32),
                                bonds_j.astype(jnp.int32))
    return jnp.stack([vx, vy, vz], axis=-1), dist


# speedup vs baseline: 30.9830x; 1.1183x over previous
import jax
import jax.numpy as jnp
from jax import lax
from jax.experimental import pallas as pl
from jax.experimental.pallas import tpu as pltpu
from jax.experimental.pallas import tpu_sc as plsc

N_ATOMS = 50000
N_EDGES = 1600000
NUM_CORES = 2
NUM_SUBCORES = 16
NW = NUM_CORES * NUM_SUBCORES
EPW = N_EDGES // NW
BLK = 2000
NBLK = EPW // BLK


def _build_body(ax_hbm, ay_hbm, az_hbm, t8_hbm, cx, cy, cz, rows8, sem):
    wid = lax.axis_index("s") * NUM_CORES + lax.axis_index("c")
    iota16 = lax.iota(jnp.int32, 16)
    col0 = iota16 * 0
    col1 = col0 + 1
    col2 = col0 + 2
    APW = N_ATOMS // 25

    @pl.when(wid < 25)
    def _():
        a0 = wid * APW
        pltpu.sync_copy(ax_hbm.at[pl.ds(a0, APW)], cx)
        pltpu.sync_copy(ay_hbm.at[pl.ds(a0, APW)], cy)
        pltpu.sync_copy(az_hbm.at[pl.ds(a0, APW)], cz)

        def grp(g, carry):
            o = g * 16
            rows = iota16 + o
            plsc.store_scatter(rows8, [rows, col0], cx[pl.ds(o, 16)])
            plsc.store_scatter(rows8, [rows, col1], cy[pl.ds(o, 16)])
            plsc.store_scatter(rows8, [rows, col2], cz[pl.ds(o, 16)])
            return carry

        lax.fori_loop(0, APW // 16, grp, 0, unroll=False)
        pltpu.sync_copy(rows8, t8_hbm.at[pl.ds(a0, APW)])


def _sc_body(ax4_hbm, bi_hbm, bj_hbm,
             ovx_hbm, ovy_hbm, ovz_hbm, odist_hbm,
             ii0, ij0, ii1, ij1,
             g0i, g0j, g1i, g1j,
             o0x, o0y, o0z, o0d, o1x, o1y, o1z, o1d,
             sg0, sg1, si0, si1, so0, so1):
    wid = lax.axis_index("s") * NUM_CORES + lax.axis_index("c")
    base = wid * EPW
    iota16 = lax.iota(jnp.int32, 16)
    col0 = iota16 * 0
    col1 = col0 + 1
    col2 = col0 + 2

    idx = [(ii0, ij0), (ii1, ij1)]
    gath = [(g0i, g0j), (g1i, g1j)]
    outs = [(o0x, o0y, o0z, o0d), (o1x, o1y, o1z, o1d)]
    sgs = [sg0, sg1]
    sis = [si0, si1]
    sos = [so0, so1]

    def issue_idx(b):
        p = b % 2
        e0 = base + b * BLK
        return (pltpu.async_copy(bi_hbm.at[pl.ds(e0, BLK)], idx[p][0], sis[p]),
                pltpu.async_copy(bj_hbm.at[pl.ds(e0, BLK)], idx[p][1], sis[p]))

    def issue_gathers(b):
        p = b % 2
        gi, gj = idx[p]
        ri, rj = gath[p]
        s = sgs[p]
        return (pltpu.async_copy(ax4_hbm.at[gi], ri, s),
                pltpu.async_copy(ax4_hbm.at[gj], rj, s))

    def issue_outs(b):
        p = b % 2
        vx_v, vy_v, vz_v, dist_v = outs[p]
        e0 = base + b * BLK
        s = sos[p]
        return (pltpu.async_copy(vx_v, ovx_hbm.at[pl.ds(e0, BLK)], s),
                pltpu.async_copy(vy_v, ovy_hbm.at[pl.ds(e0, BLK)], s),
                pltpu.async_copy(vz_v, ovz_hbm.at[pl.ds(e0, BLK)], s),
                pltpu.async_copy(dist_v, odist_hbm.at[pl.ds(e0, BLK)], s))

    def compute(b):
        p = b % 2
        ri, rj = gath[p]
        vx_v, vy_v, vz_v, dist_v = outs[p]

        def group_body(g, carry):
            o = g * 16
            rows = iota16 + o
            v0 = plsc.load_gather(rj, [rows, col0]) - plsc.load_gather(ri, [rows, col0])
            v1 = plsc.load_gather(rj, [rows, col1]) - plsc.load_gather(ri, [rows, col1])
            v2 = plsc.load_gather(rj, [rows, col2]) - plsc.load_gather(ri, [rows, col2])
            vx_v[pl.ds(o, 16)] = v0
            vy_v[pl.ds(o, 16)] = v1
            vz_v[pl.ds(o, 16)] = v2
            d2 = v0 * v0 + v1 * v1 + v2 * v2
            d2c = jnp.maximum(d2, 1.1754944e-38)
            y = plsc.bitcast(0x5F3759DF - (plsc.bitcast(d2c, jnp.int32) >> 1),
                             jnp.float32)
            y = y * (1.5 - 0.5 * d2c * y * y)
            y = y * (1.5 - 0.5 * d2c * y * y)
            y = y * (1.5 - 0.5 * d2c * y * y)
            dist_v[pl.ds(o, 16)] = d2 * y
            return carry

        lax.fori_loop(0, BLK // 16, group_body, 0, unroll=False)

    idx_descs = {0: issue_idx(0)}
    for d in idx_descs[0]:
        d.wait()
    g_descs = {0: issue_gathers(0)}
    idx_descs[1] = issue_idx(1)
    out_descs = {}
    for b in range(NBLK):
        if b + 1 < NBLK:
            for d in idx_descs[b + 1]:
                d.wait()
            g_descs[b + 1] = issue_gathers(b + 1)
        for d in g_descs[b]:
            d.wait()
        if b + 2 < NBLK:
            idx_descs[b + 2] = issue_idx(b + 2)
        if b - 2 in out_descs:
            for d in out_descs[b - 2]:
                d.wait()
        compute(b)
        out_descs[b] = issue_outs(b)
    for d in out_descs[NBLK - 2]:
        d.wait()
    for d in out_descs[NBLK - 1]:
        d.wait()


@jax.jit
def _sc_call(ax, ay, az, bi, bj):
    mesh = plsc.VectorSubcoreMesh(core_axis_name="c", subcore_axis_name="s",
                                  num_cores=NUM_CORES,
                                  num_subcores=NUM_SUBCORES)
    build = pl.kernel(
        _build_body,
        out_type=jax.ShapeDtypeStruct((N_ATOMS, 8), jnp.float32),
        mesh=mesh,
        compiler_params=pltpu.CompilerParams(needs_layout_passes=False,
                                             use_tc_tiling_on_sc=False),
        scratch_types=[
            pltpu.VMEM((N_ATOMS // 25,), jnp.float32),
            pltpu.VMEM((N_ATOMS // 25,), jnp.float32),
            pltpu.VMEM((N_ATOMS // 25,), jnp.float32),
            pltpu.VMEM((N_ATOMS // 25, 8), jnp.float32),
            pltpu.SemaphoreType.DMA,
        ],
    )
    ax4 = build(ax, ay, az)
    fvec = pltpu.VMEM((BLK,), jnp.float32)
    ivec = pltpu.VMEM((BLK,), jnp.int32)
    rvec = pltpu.VMEM((BLK, 8), jnp.float32)
    f = pl.kernel(
        _sc_body,
        out_type=(jax.ShapeDtypeStruct((N_EDGES,), jnp.float32),
                  jax.ShapeDtypeStruct((N_EDGES,), jnp.float32),
                  jax.ShapeDtypeStruct((N_EDGES,), jnp.float32),
                  jax.ShapeDtypeStruct((N_EDGES,), jnp.float32)),
        mesh=mesh,
        compiler_params=pltpu.CompilerParams(needs_layout_passes=False,
                                             use_tc_tiling_on_sc=False),
        scratch_types=(
            [ivec] * 4 + [rvec] * 4 + [fvec] * 8
            + [pltpu.SemaphoreType.DMA] * 6
        ),
    )
    return f(ax4, bi, bj)


def kernel(atoms_x, bonds_i, bonds_j):
    vx, vy, vz, dist = _sc_call(atoms_x[:, 0], atoms_x[:, 1], atoms_x[:, 2],
                                bonds_i.astype(jnp.int32),
                                bonds_j.astype(jnp.int32))
    return jnp.stack([vx, vy, vz], axis=-1), dist


# speedup vs baseline: 47.4469x; 1.5314x over previous
import jax
import jax.numpy as jnp
from jax import lax
from jax.experimental import pallas as pl
from jax.experimental.pallas import tpu as pltpu
from jax.experimental.pallas import tpu_sc as plsc

N_ATOMS = 50000
N_EDGES = 1600000
NUM_CORES = 2
NUM_SUBCORES = 16
NW = NUM_CORES * NUM_SUBCORES
NCHUNK = N_EDGES // 128
CPW = 390
NTAIL = NCHUNK - NW * CPW
BLKC = 15
BLKE = BLKC * 128
NBLK = CPW // BLKC
APW = N_ATOMS // 25


def _build_body(ax_hbm, ay_hbm, az_hbm, t8_hbm, cx, cy, cz, rows8, sem):
    wid = lax.axis_index("s") * NUM_CORES + lax.axis_index("c")
    iota16 = lax.iota(jnp.int32, 16)
    col0 = iota16 * 0
    col1 = col0 + 1
    col2 = col0 + 2

    @pl.when(wid < 25)
    def _():
        a0 = wid * APW
        pltpu.sync_copy(ax_hbm.at[pl.ds(a0, APW)], cx)
        pltpu.sync_copy(ay_hbm.at[pl.ds(a0, APW)], cy)
        pltpu.sync_copy(az_hbm.at[pl.ds(a0, APW)], cz)

        def grp(g, carry):
            o = g * 16
            rows = iota16 + o
            plsc.store_scatter(rows8, [rows, col0], cx[pl.ds(o, 16)])
            plsc.store_scatter(rows8, [rows, col1], cy[pl.ds(o, 16)])
            plsc.store_scatter(rows8, [rows, col2], cz[pl.ds(o, 16)])
            return carry

        lax.fori_loop(0, APW // 16, grp, 0, unroll=False)
        pltpu.sync_copy(rows8, t8_hbm.at[pl.ds(a0, APW)])


def _sc_body(t8_hbm, bi_hbm, bj_hbm, ovec_hbm, odist_hbm,
             ii0, ij0, ii1, ij1,
             g0i, g0j, g1i, g1j,
             o0v, o0d, o1v, o1d,
             sg0, sg1, si0, si1, so0, so1):
    wid = lax.axis_index("s") * NUM_CORES + lax.axis_index("c")
    q_base = wid * CPW
    iota16 = lax.iota(jnp.int32, 16)
    col0 = iota16 * 0
    col1 = col0 + 1
    col2 = col0 + 2

    idx = [(ii0, ij0), (ii1, ij1)]
    gath = [(g0i, g0j), (g1i, g1j)]
    outs = [(o0v, o0d), (o1v, o1d)]
    sgs = [sg0, sg1]
    sis = [si0, si1]
    sos = [so0, so1]

    def issue_idx(b):
        p = b % 2
        e0 = (q_base + b * BLKC) * 128
        return (pltpu.async_copy(bi_hbm.at[pl.ds(e0, BLKE)], idx[p][0], sis[p]),
                pltpu.async_copy(bj_hbm.at[pl.ds(e0, BLKE)], idx[p][1], sis[p]))

    def issue_gathers(b):
        p = b % 2
        gi, gj = idx[p]
        ri, rj = gath[p]
        s = sgs[p]
        return (pltpu.async_copy(t8_hbm.at[gi], ri, s),
                pltpu.async_copy(t8_hbm.at[gj], rj, s))

    def issue_outs(b):
        p = b % 2
        vec_v, dist_v = outs[p]
        q0 = q_base + b * BLKC
        s = sos[p]
        return (pltpu.async_copy(vec_v, ovec_hbm.at[pl.ds(q0 * 512, BLKC * 512)], s),
                pltpu.async_copy(dist_v, odist_hbm.at[pl.ds(q0 * 128, BLKE)], s))

    def compute(b):
        p = b % 2
        ri, rj = gath[p]
        vec_v, dist_v = outs[p]

        def group_body(g, carry):
            o = g * 16
            rows = iota16 + o
            v0 = plsc.load_gather(rj, [rows, col0]) - plsc.load_gather(ri, [rows, col0])
            v1 = plsc.load_gather(rj, [rows, col1]) - plsc.load_gather(ri, [rows, col1])
            v2 = plsc.load_gather(rj, [rows, col2]) - plsc.load_gather(ri, [rows, col2])
            vb = (g >> 3) * 512 + (g & 7) * 16
            vec_v[pl.ds(vb, 16)] = v0
            vec_v[pl.ds(vb + 128, 16)] = v1
            vec_v[pl.ds(vb + 256, 16)] = v2
            d2 = v0 * v0 + v1 * v1 + v2 * v2
            d2c = jnp.maximum(d2, 1.1754944e-38)
            y = plsc.bitcast(0x5F3759DF - (plsc.bitcast(d2c, jnp.int32) >> 1),
                             jnp.float32)
            y = y * (1.5 - 0.5 * d2c * y * y)
            y = y * (1.5 - 0.5 * d2c * y * y)
            y = y * (1.5 - 0.5 * d2c * y * y)
            dist_v[pl.ds(o, 16)] = d2 * y
            return carry

        lax.fori_loop(0, BLKE // 16, group_body, 0, unroll=False)

    idx_descs = {0: issue_idx(0)}
    for d in idx_descs[0]:
        d.wait()
    g_descs = {0: issue_gathers(0)}
    idx_descs[1] = issue_idx(1)
    out_descs = {}
    for b in range(NBLK):
        if b + 1 < NBLK:
            for d in idx_descs[b + 1]:
                d.wait()
            g_descs[b + 1] = issue_gathers(b + 1)
        for d in g_descs[b]:
            d.wait()
        if b + 2 < NBLK:
            idx_descs[b + 2] = issue_idx(b + 2)
        if b - 2 in out_descs:
            for d in out_descs[b - 2]:
                d.wait()
        compute(b)
        out_descs[b] = issue_outs(b)
    for d in out_descs[NBLK - 2]:
        d.wait()
    for d in out_descs[NBLK - 1]:
        d.wait()

    @pl.when(wid < NTAIL)
    def _():
        qt = NW * CPW + wid
        et = qt * 128
        gi = idx[0][0].at[pl.ds(0, 128)]
        gj = idx[0][1].at[pl.ds(0, 128)]
        pltpu.sync_copy(bi_hbm.at[pl.ds(et, 128)], gi)
        pltpu.sync_copy(bj_hbm.at[pl.ds(et, 128)], gj)
        ri = gath[0][0].at[pl.ds(0, 128)]
        rj = gath[0][1].at[pl.ds(0, 128)]
        c1 = pltpu.async_copy(t8_hbm.at[gi], ri, sg0)
        c2 = pltpu.async_copy(t8_hbm.at[gj], rj, sg0)
        c1.wait()
        c2.wait()
        vec_v, dist_v = outs[0]

        def tail_group(g, carry):
            o = g * 16
            rows = iota16 + o
            u0 = plsc.load_gather(rj, [rows, col0]) - plsc.load_gather(ri, [rows, col0])
            u1 = plsc.load_gather(rj, [rows, col1]) - plsc.load_gather(ri, [rows, col1])
            u2 = plsc.load_gather(rj, [rows, col2]) - plsc.load_gather(ri, [rows, col2])
            vec_v[pl.ds(o, 16)] = u0
            vec_v[pl.ds(o + 128, 16)] = u1
            vec_v[pl.ds(o + 256, 16)] = u2
            d2 = u0 * u0 + u1 * u1 + u2 * u2
            d2c = jnp.maximum(d2, 1.1754944e-38)
            y = plsc.bitcast(0x5F3759DF - (plsc.bitcast(d2c, jnp.int32) >> 1),
                             jnp.float32)
            y = y * (1.5 - 0.5 * d2c * y * y)
            y = y * (1.5 - 0.5 * d2c * y * y)
            y = y * (1.5 - 0.5 * d2c * y * y)
            dist_v[pl.ds(o, 16)] = d2 * y
            return carry

        lax.fori_loop(0, 8, tail_group, 0, unroll=False)
        pltpu.sync_copy(vec_v.at[pl.ds(0, 512)],
                        ovec_hbm.at[pl.ds(qt * 512, 512)])
        pltpu.sync_copy(dist_v.at[pl.ds(0, 128)],
                        odist_hbm.at[pl.ds(et, 128)])


@jax.jit
def _sc_call(ax, ay, az, bi, bj):
    mesh = plsc.VectorSubcoreMesh(core_axis_name="c", subcore_axis_name="s",
                                  num_cores=NUM_CORES,
                                  num_subcores=NUM_SUBCORES)
    cparams = pltpu.CompilerParams(needs_layout_passes=False,
                                   use_tc_tiling_on_sc=False)
    build = pl.kernel(
        _build_body,
        out_type=jax.ShapeDtypeStruct((N_ATOMS, 8), jnp.float32),
        mesh=mesh,
        compiler_params=cparams,
        scratch_types=[
            pltpu.VMEM((APW,), jnp.float32),
            pltpu.VMEM((APW,), jnp.float32),
            pltpu.VMEM((APW,), jnp.float32),
            pltpu.VMEM((APW, 8), jnp.float32),
            pltpu.SemaphoreType.DMA,
        ],
    )
    t8 = build(ax, ay, az)
    ivec = pltpu.VMEM((BLKE,), jnp.int32)
    rvec = pltpu.VMEM((BLKE, 8), jnp.float32)
    vvec = pltpu.VMEM((BLKC * 512,), jnp.float32)
    dvec = pltpu.VMEM((BLKE,), jnp.float32)
    f = pl.kernel(
        _sc_body,
        out_type=(jax.ShapeDtypeStruct((NCHUNK * 512,), jnp.float32),
                  jax.ShapeDtypeStruct((N_EDGES,), jnp.float32)),
        mesh=mesh,
        compiler_params=cparams,
        scratch_types=(
            [ivec] * 4 + [rvec] * 4 + [vvec, dvec, vvec, dvec]
            + [pltpu.SemaphoreType.DMA] * 6
        ),
    )
    return f(t8, bi, bj)


def kernel(atoms_x, bonds_i, bonds_j):
    vec_raw, dist = _sc_call(atoms_x[:, 0], atoms_x[:, 1], atoms_x[:, 2],
                             bonds_i.astype(jnp.int32),
                             bonds_j.astype(jnp.int32))
    bond_vec = (vec_raw.reshape(NCHUNK, 4, 128)
                .transpose(0, 2, 1)
                .reshape(N_EDGES, 4)[:, :3])
    return bond_vec, dist
